# baseline (speedup 1.0000x reference)
import jax
import jax.numpy as jnp
from jax import lax
from jax.experimental import pallas as pl
from jax.experimental.pallas import tpu as pltpu
from jax.experimental.pallas import tpu_sc as plsc

VOCAB = 1000000
MAXLEN = 200
EMBED_DIM = 32
BATCH = 4096
SEQ = 200

NC = 2
NS = 16
NW = NC * NS

BB = BATCH // NW
NBUF = 8
LOOK = 4


def _sc_kernel(xT_hbm, tok_hbm, pos_hbm, out_hbm, idx_v, g_v, pos_v, *sems):
    sem_g = sems[:NBUF]
    sem_s = sems[NBUF:]
    wid = lax.axis_index("s") * NC + lax.axis_index("c")
    b0 = wid * BB

    pltpu.sync_copy(pos_hbm, pos_v)
    pltpu.sync_copy(xT_hbm.at[:, pl.ds(b0, BB)], idx_v)

    def fire_gather(s, b):
        pltpu.async_copy(tok_hbm.at[idx_v.at[s]], g_v.at[b], sem_g[b])

    def drain_gather(s, b):
        pltpu.make_async_copy(tok_hbm.at[idx_v.at[s]], g_v.at[b],
                              sem_g[b]).wait()

    def compute(s, b):
        pv = [pos_v[s, pl.ds(k * 16, 16)] for k in range(2)]

        def r_body(rr, acc):
            for u in range(8):
                r = rr * 8 + u
                for k in range(2):
                    plsc.addupdate(g_v.at[b, r, pl.ds(k * 16, 16)], pv[k])
            return acc
        lax.fori_loop(0, BB // 8, r_body, 0)

    def fire_out(s, b):
        pltpu.async_copy(g_v.at[b], out_hbm.at[s, wid], sem_s[b])

    def drain_out(s, b):
        pltpu.make_async_copy(g_v.at[b], out_hbm.at[s, wid], sem_s[b]).wait()

    for s in range(LOOK):
        fire_gather(s, s)

    def main_body(it, carry):
        for j in range(NBUF):
            s = it * NBUF + j
            bn = (j + LOOK) % NBUF

            @pl.when(s >= NBUF - LOOK)
            def _():
                drain_out(s - (NBUF - LOOK), bn)

            @pl.when(s < SEQ - LOOK)
            def _():
                fire_gather(s + LOOK, bn)

            drain_gather(s, j)
            compute(s, j)
            fire_out(s, j)
        return carry
    lax.fori_loop(0, SEQ // NBUF, main_body, 0)

    for s in range(SEQ - (NBUF - LOOK), SEQ):
        drain_out(s, s % NBUF)


def _tc_transpose_kernel(in_ref, out_ref):
    v = in_ref[0].reshape(256, 4, EMBED_DIM)
    out_ref[0] = v.transpose(2, 0, 1).reshape(EMBED_DIM, 1024)


def kernel(x, token_table, pos_table):
    xT = jnp.swapaxes(x, 0, 1).astype(jnp.int32)
    mesh = plsc.VectorSubcoreMesh(core_axis_name="c", subcore_axis_name="s",
                                  num_cores=NC, num_subcores=NS)
    out4 = pl.kernel(
        _sc_kernel,
        out_type=jax.ShapeDtypeStruct((SEQ, NW, BB, EMBED_DIM), jnp.float32),
        mesh=mesh,
        compiler_params=pltpu.CompilerParams(use_tc_tiling_on_sc=False),
        scratch_types=[
            pltpu.VMEM((SEQ, BB), jnp.int32),
            pltpu.VMEM((NBUF, BB, EMBED_DIM), jnp.float32),
            pltpu.VMEM((MAXLEN, EMBED_DIM), jnp.float32),
        ] + [pltpu.SemaphoreType.DMA] * (2 * NBUF),
    )(xT, token_table, pos_table)
    lin = out4.reshape(SEQ, BATCH * EMBED_DIM // 128, 128)
    out_t = pl.pallas_call(
        _tc_transpose_kernel,
        grid=(SEQ, 4),
        in_specs=[pl.BlockSpec((1, 256, 128), lambda s, b4: (s, b4, 0))],
        out_specs=pl.BlockSpec((1, EMBED_DIM, 1024), lambda s, b4: (s, 0, b4)),
        out_shape=jax.ShapeDtypeStruct((SEQ, EMBED_DIM, BATCH), jnp.float32),
    )(lin)
    return out_t.transpose(2, 0, 1)


# speedup vs baseline: 3.2778x; 3.2778x over previous
import jax
import jax.numpy as jnp
from jax import lax
from jax.experimental import pallas as pl
from jax.experimental.pallas import tpu as pltpu
from jax.experimental.pallas import tpu_sc as plsc

VOCAB = 1000000
MAXLEN = 200
EMBED_DIM = 32
BATCH = 4096
SEQ = 200

NC = 2
NS = 16
NW = NC * NS

BB = BATCH // NW
NBUF = 8
LOOK = 4


def _sc_kernel(xT_hbm, tok_hbm, pos_hbm, out_hbm, idx_v, g_v, pos_v, *sems):
    sem_g = sems[:NBUF]
    sem_s = sems[NBUF:]
    wid = lax.axis_index("s") * NC + lax.axis_index("c")
    b0 = wid * BB

    pltpu.sync_copy(pos_hbm, pos_v)
    pltpu.sync_copy(xT_hbm.at[:, pl.ds(b0, BB)], idx_v)

    def fire_gather(s, b):
        pltpu.async_copy(tok_hbm.at[idx_v.at[s]], g_v.at[b], sem_g[b])

    def drain_gather(s, b):
        pltpu.make_async_copy(tok_hbm.at[idx_v.at[s]], g_v.at[b],
                              sem_g[b]).wait()

    def compute(s, b):
        pv = [pos_v[s, pl.ds(k * 16, 16)] for k in range(2)]

        def r_body(rr, acc):
            for u in range(8):
                r = rr * 8 + u
                for k in range(2):
                    plsc.addupdate(g_v.at[b, r, pl.ds(k * 16, 16)], pv[k])
            return acc
        lax.fori_loop(0, BB // 8, r_body, 0)

    def fire_out(s, b):
        pltpu.async_copy(g_v.at[b], out_hbm.at[s, wid], sem_s[b])

    def drain_out(s, b):
        pltpu.make_async_copy(g_v.at[b], out_hbm.at[s, wid], sem_s[b]).wait()

    for s in range(LOOK):
        fire_gather(s, s)

    def main_body(it, carry):
        for j in range(NBUF):
            s = it * NBUF + j
            bn = (j + LOOK) % NBUF

            @pl.when(s >= NBUF - LOOK)
            def _():
                drain_out(s - (NBUF - LOOK), bn)

            @pl.when(s < SEQ - LOOK)
            def _():
                fire_gather(s + LOOK, bn)

            drain_gather(s, j)
            compute(s, j)
            fire_out(s, j)
        return carry
    lax.fori_loop(0, SEQ // NBUF, main_body, 0)

    for s in range(SEQ - (NBUF - LOOK), SEQ):
        drain_out(s, s % NBUF)


def kernel(x, token_table, pos_table):
    xT = jnp.swapaxes(x, 0, 1).astype(jnp.int32)
    mesh = plsc.VectorSubcoreMesh(core_axis_name="c", subcore_axis_name="s",
                                  num_cores=NC, num_subcores=NS)
    out4 = pl.kernel(
        _sc_kernel,
        out_type=jax.ShapeDtypeStruct((SEQ, NW, BB, EMBED_DIM), jnp.float32),
        mesh=mesh,
        compiler_params=pltpu.CompilerParams(use_tc_tiling_on_sc=False),
        scratch_types=[
            pltpu.VMEM((SEQ, BB), jnp.int32),
            pltpu.VMEM((NBUF, BB, EMBED_DIM), jnp.float32),
            pltpu.VMEM((MAXLEN, EMBED_DIM), jnp.float32),
        ] + [pltpu.SemaphoreType.DMA] * (2 * NBUF),
    )(xT, token_table, pos_table)
    return out4.reshape(SEQ, BATCH, EMBED_DIM).transpose(1, 0, 2)
